# baseline (speedup 1.0000x reference)
import functools

import jax
import jax.numpy as jnp
from jax import lax
from jax.experimental import pallas as pl
from jax.experimental.pallas import tpu as pltpu
from jax.experimental.pallas import tpu_sc as plsc

N_NODES = 10000
N_EDGES = 320000
DIM = 128
PDIM = DIM // 2

NC = 2
NS = 16
NW = NC * NS

CHUNK = 64
CH_PER_TILE = 160
E_PAD = NW * CH_PER_TILE * CHUNK
N_PAD = 10112
ROWS_PER_TILE = N_PAD // NS
FLUSH_FULL = ROWS_PER_TILE // CHUNK
FLUSH_REM = ROWS_PER_TILE - FLUSH_FULL * CHUNK


def _sc_body(idx_hbm, featp_hbm, out_hbm,
             rows0, rows1, pb0, pb1, ix0, ix1, ix2, ix3,
             sg0, sg1, ss0, ss1, si0, si1, si2, si3, acc_sh):
    rows = [rows0, rows1]
    pb = [pb0, pb1]
    ix = [ix0, ix1, ix2, ix3]
    sg = [sg0, sg1]
    ss = [ss0, ss1]
    si = [si0, si1, si2, si3]
    c = lax.axis_index("c")
    s = lax.axis_index("s")
    wid = c * NS + s
    base = wid * CH_PER_TILE

    def _issue_idx(j, q):
        pltpu.async_copy(idx_hbm.at[base + j], ix[q], si[q])

    def _wait_idx(q):
        pltpu.make_async_copy(idx_hbm.at[base], ix[q], si[q]).wait()

    def _issue_gather(q, t):
        pltpu.async_copy(featp_hbm.at[ix[q].at[0]], pb[t], sg[t])

    def _wait_gather(t):
        pltpu.make_async_copy(featp_hbm.at[ix0.at[0]], pb[t], sg[t]).wait()

    def _issue_scatter(q, t):
        pltpu.async_copy(rows[t], acc_sh.at[ix[q].at[1]], ss[t], add=True)

    def _wait_scatter(t):
        pltpu.make_async_copy(rows[t], acc_sh.at[ix0.at[1]], ss[t]).wait()

    def _scale(t, q):
        src = pb[t]
        dst = rows[t]
        vref = ix[q]
        himask = jnp.full((16,), -65536, jnp.int32)

        def _grp(g, ecarry):
            vv = lax.bitcast_convert_type(
                vref[2, pl.ds(g * 16, 16)], jnp.float32)
            for l in range(16):
                vb = jnp.full((16,), vv[l], jnp.float32)
                e = g * 16 + l
                for w in range(PDIM // 16):
                    xi = src[e, pl.ds(w * 16, 16)]
                    a = lax.bitcast_convert_type(xi << 16, jnp.float32)
                    b = lax.bitcast_convert_type(xi & himask, jnp.float32)
                    dst[e, pl.ds(w * 32, 16)] = a * vb
                    dst[e, pl.ds(w * 32 + 16, 16)] = b * vb
            return ecarry

        lax.fori_loop(0, CHUNK // 16, _grp, 0)

    zero = jnp.zeros((16,), jnp.float32)

    def _zrow(i, carry):
        for k in range(DIM // 16):
            rows0[i, pl.ds(k * 16, 16)] = zero
        return carry

    lax.fori_loop(0, CHUNK, _zrow, 0)
    acc_base = s * ROWS_PER_TILE

    def _seg(k):
        r0 = acc_base + k * CHUNK
        n = CHUNK if k < FLUSH_FULL else FLUSH_REM
        return acc_sh.at[pl.ds(r0, n)], n

    for k in range(FLUSH_FULL + 1):
        dst, n = _seg(k)
        pltpu.async_copy(rows0.at[pl.ds(0, n)], dst, sg0)
    for k in range(FLUSH_FULL + 1):
        dst, n = _seg(k)
        pltpu.make_async_copy(rows0.at[pl.ds(0, n)], dst, sg0).wait()
    plsc.subcore_barrier()

    _issue_idx(0, 0)
    _issue_idx(1, 1)
    _wait_idx(0)
    _issue_gather(0, 0)

    def _outer(kk, carry):
        for b in range(4):
            j = kk * 4 + b
            t = b % 2
            _wait_gather(t)

            @pl.when(j >= 1)
            def _(t=t):
                _wait_scatter(1 - t)

            @pl.when(j + 2 < CH_PER_TILE)
            def _(j=j, b=b):
                _issue_idx(j + 2, (b + 2) % 4)

            @pl.when(j + 1 < CH_PER_TILE)
            def _(b=b, t=t):
                _wait_idx((b + 1) % 4)
                _issue_gather((b + 1) % 4, 1 - t)

            _scale(t, b)
            _issue_scatter(b, t)
        return carry

    lax.fori_loop(0, CH_PER_TILE // 4, _outer, 0)
    _wait_scatter((CH_PER_TILE - 1) % 2)
    plsc.subcore_barrier()

    NSEG = FLUSH_FULL + 1

    def _in_copy(k, t, wait):
        r0 = acc_base + k * CHUNK
        n = CHUNK if k < FLUSH_FULL else FLUSH_REM
        args = (acc_sh.at[pl.ds(r0, n)], rows[t].at[pl.ds(0, n)], sg[t])
        if wait:
            pltpu.make_async_copy(*args).wait()
        else:
            pltpu.async_copy(*args)

    def _out_copy(k, t, wait):
        r0 = acc_base + k * CHUNK
        n = CHUNK if k < FLUSH_FULL else FLUSH_REM
        args = (rows[t].at[pl.ds(0, n)], out_hbm.at[c, pl.ds(r0, n)], ss[t])
        if wait:
            pltpu.make_async_copy(*args).wait()
        else:
            pltpu.async_copy(*args)

    _in_copy(0, 0, False)
    for k in range(NSEG):
        t = k % 2
        _in_copy(k, t, True)
        if k + 1 < NSEG:
            if k >= 1:
                _out_copy(k - 1, 1 - t, True)
            _in_copy(k + 1, 1 - t, False)
        _out_copy(k, t, False)
    _out_copy(NSEG - 2, NSEG % 2, True)
    _out_copy(NSEG - 1, (NSEG - 1) % 2, True)


_sc_segment_sum = functools.partial(
    pl.kernel,
    out_type=jax.ShapeDtypeStruct((NC, N_PAD, DIM), jnp.float32),
    mesh=plsc.VectorSubcoreMesh(core_axis_name="c", subcore_axis_name="s"),
    compiler_params=pltpu.CompilerParams(use_tc_tiling_on_sc=False),
    scratch_types=[
        pltpu.VMEM((CHUNK, DIM), jnp.float32),
        pltpu.VMEM((CHUNK, DIM), jnp.float32),
        pltpu.VMEM((CHUNK, PDIM), jnp.int32),
        pltpu.VMEM((CHUNK, PDIM), jnp.int32),
        pltpu.VMEM((3, CHUNK), jnp.int32),
        pltpu.VMEM((3, CHUNK), jnp.int32),
        pltpu.VMEM((3, CHUNK), jnp.int32),
        pltpu.VMEM((3, CHUNK), jnp.int32),
        pltpu.SemaphoreType.DMA,
        pltpu.SemaphoreType.DMA,
        pltpu.SemaphoreType.DMA,
        pltpu.SemaphoreType.DMA,
        pltpu.SemaphoreType.DMA,
        pltpu.SemaphoreType.DMA,
        pltpu.SemaphoreType.DMA,
        pltpu.SemaphoreType.DMA,
        pltpu.VMEM_SHARED((N_PAD, DIM), jnp.float32),
    ],
)(_sc_body)


def _tc_body(f_ref, p0_ref, p1_ref, w1_ref, w2_ref, b1_ref, b2_ref, o_ref):
    f = f_ref[...]
    h = p0_ref[...] + p1_ref[...]
    a = lax.dot_general(f + h, w1_ref[...], (((1,), (1,)), ((), ())),
                        preferred_element_type=jnp.float32)
    b = lax.dot_general(f * h, w2_ref[...], (((1,), (1,)), ((), ())),
                        preferred_element_type=jnp.float32)
    x = a + b + b1_ref[...] + b2_ref[...]
    o_ref[...] = jnp.where(x > 0, x, 0.01 * x)


def _tc_mlp(features, p0, p1, W1_w, W2_w, b1, b2):
    block = 2000
    grid = N_NODES // block
    row_spec = pl.BlockSpec((block, DIM), lambda i: (i, 0))
    full_spec = pl.BlockSpec((DIM, DIM), lambda i: (0, 0))
    bias_spec = pl.BlockSpec((1, DIM), lambda i: (0, 0))
    return pl.pallas_call(
        _tc_body,
        grid=(grid,),
        in_specs=[row_spec, row_spec, row_spec, full_spec, full_spec,
                  bias_spec, bias_spec],
        out_specs=row_spec,
        out_shape=jax.ShapeDtypeStruct((N_NODES, DIM), jnp.float32),
    )(features, p0, p1, W1_w, W2_w, b1, b2)


def kernel(features, target, neighbor, values, W1_w, W1_b, W2_w, W2_b):
    pad = E_PAD - N_EDGES
    nbr = jnp.concatenate(
        [neighbor.astype(jnp.int32), jnp.zeros((pad,), jnp.int32)]
    ).reshape(E_PAD // CHUNK, CHUNK)
    tgt = jnp.concatenate(
        [target.astype(jnp.int32), jnp.zeros((pad,), jnp.int32)]
    ).reshape(E_PAD // CHUNK, CHUNK)
    val = lax.bitcast_convert_type(
        jnp.concatenate(
            [values.astype(jnp.float32), jnp.zeros((pad,), jnp.float32)]
        ), jnp.int32
    ).reshape(E_PAD // CHUNK, CHUNK)
    idx = jnp.stack([nbr, tgt, val], axis=1)

    featb = features.astype(jnp.bfloat16).reshape(N_NODES, 4, 2, 16)
    featp = lax.bitcast_convert_type(
        featb.transpose(0, 1, 3, 2), jnp.int32).reshape(N_NODES, PDIM)

    partials = _sc_segment_sum(idx, featp)
    return _tc_mlp(features, partials[0, :N_NODES], partials[1, :N_NODES],
                   W1_w, W2_w, W1_b.reshape(1, DIM), W2_b.reshape(1, DIM))


# speedup vs baseline: 1.0274x; 1.0274x over previous
import functools

import jax
import jax.numpy as jnp
from jax import lax
from jax.experimental import pallas as pl
from jax.experimental.pallas import tpu as pltpu
from jax.experimental.pallas import tpu_sc as plsc

N_NODES = 10000
N_EDGES = 320000
DIM = 128
PDIM = DIM // 2

NC = 2
NS = 16
NW = NC * NS

CHUNK = 80
CH_PER_TILE = 128
E_PAD = NW * CH_PER_TILE * CHUNK
N_PAD = 10112
ROWS_PER_TILE = N_PAD // NS
FLUSH_FULL = ROWS_PER_TILE // CHUNK
FLUSH_REM = ROWS_PER_TILE - FLUSH_FULL * CHUNK


def _sc_body(idx_hbm, featp_hbm, out_hbm,
             rows0, rows1, pb0, pb1, ix0, ix1, ix2, ix3,
             sg0, sg1, ss0, ss1, si0, si1, si2, si3, acc_sh):
    rows = [rows0, rows1]
    pb = [pb0, pb1]
    ix = [ix0, ix1, ix2, ix3]
    sg = [sg0, sg1]
    ss = [ss0, ss1]
    si = [si0, si1, si2, si3]
    c = lax.axis_index("c")
    s = lax.axis_index("s")
    wid = c * NS + s
    base = wid * CH_PER_TILE

    def _issue_idx(j, q):
        pltpu.async_copy(idx_hbm.at[base + j], ix[q], si[q])

    def _wait_idx(q):
        pltpu.make_async_copy(idx_hbm.at[base], ix[q], si[q]).wait()

    def _issue_gather(q, t):
        pltpu.async_copy(featp_hbm.at[ix[q].at[0]], pb[t], sg[t])

    def _wait_gather(t):
        pltpu.make_async_copy(featp_hbm.at[ix0.at[0]], pb[t], sg[t]).wait()

    def _issue_scatter(q, t):
        pltpu.async_copy(rows[t], acc_sh.at[ix[q].at[1]], ss[t], add=True)

    def _wait_scatter(t):
        pltpu.make_async_copy(rows[t], acc_sh.at[ix0.at[1]], ss[t]).wait()

    def _scale(t, q):
        src = pb[t]
        dst = rows[t]
        vref = ix[q]
        himask = jnp.full((16,), -65536, jnp.int32)

        def _grp(g, ecarry):
            vv = lax.bitcast_convert_type(
                vref[2, pl.ds(g * 16, 16)], jnp.float32)
            for l in range(16):
                vb = jnp.full((16,), vv[l], jnp.float32)
                e = g * 16 + l
                for w in range(PDIM // 16):
                    xi = src[e, pl.ds(w * 16, 16)]
                    a = lax.bitcast_convert_type(xi << 16, jnp.float32)
                    b = lax.bitcast_convert_type(xi & himask, jnp.float32)
                    dst[e, pl.ds(w * 32, 16)] = a * vb
                    dst[e, pl.ds(w * 32 + 16, 16)] = b * vb
            return ecarry

        lax.fori_loop(0, CHUNK // 16, _grp, 0)

    zero = jnp.zeros((16,), jnp.float32)

    def _zrow(i, carry):
        for k in range(DIM // 16):
            rows0[i, pl.ds(k * 16, 16)] = zero
        return carry

    lax.fori_loop(0, CHUNK, _zrow, 0)
    acc_base = s * ROWS_PER_TILE

    def _seg(k):
        r0 = acc_base + k * CHUNK
        n = CHUNK if k < FLUSH_FULL else FLUSH_REM
        return acc_sh.at[pl.ds(r0, n)], n

    for k in range(FLUSH_FULL + 1):
        dst, n = _seg(k)
        pltpu.async_copy(rows0.at[pl.ds(0, n)], dst, sg0)
    for k in range(FLUSH_FULL + 1):
        dst, n = _seg(k)
        pltpu.make_async_copy(rows0.at[pl.ds(0, n)], dst, sg0).wait()
    plsc.subcore_barrier()

    _issue_idx(0, 0)
    _issue_idx(1, 1)
    _wait_idx(0)
    _issue_gather(0, 0)

    def _outer(kk, carry):
        for b in range(4):
            j = kk * 4 + b
            t = b % 2
            _wait_gather(t)

            @pl.when(j + 2 < CH_PER_TILE)
            def _(j=j, b=b):
                _issue_idx(j + 2, (b + 2) % 4)

            @pl.when(j + 1 < CH_PER_TILE)
            def _(b=b, t=t):
                _wait_idx((b + 1) % 4)
                _issue_gather((b + 1) % 4, 1 - t)

            @pl.when(j >= 1)
            def _(t=t):
                _wait_scatter(1 - t)

            _scale(t, b)
            _issue_scatter(b, t)
        return carry

    lax.fori_loop(0, CH_PER_TILE // 4, _outer, 0)
    _wait_scatter((CH_PER_TILE - 1) % 2)
    plsc.subcore_barrier()

    NSEG = FLUSH_FULL + 1

    def _in_copy(k, t, wait):
        r0 = acc_base + k * CHUNK
        n = CHUNK if k < FLUSH_FULL else FLUSH_REM
        args = (acc_sh.at[pl.ds(r0, n)], rows[t].at[pl.ds(0, n)], sg[t])
        if wait:
            pltpu.make_async_copy(*args).wait()
        else:
            pltpu.async_copy(*args)

    def _out_copy(k, t, wait):
        r0 = acc_base + k * CHUNK
        n = CHUNK if k < FLUSH_FULL else FLUSH_REM
        args = (rows[t].at[pl.ds(0, n)], out_hbm.at[c, pl.ds(r0, n)], ss[t])
        if wait:
            pltpu.make_async_copy(*args).wait()
        else:
            pltpu.async_copy(*args)

    _in_copy(0, 0, False)
    for k in range(NSEG):
        t = k % 2
        _in_copy(k, t, True)
        if k + 1 < NSEG:
            if k >= 1:
                _out_copy(k - 1, 1 - t, True)
            _in_copy(k + 1, 1 - t, False)
        _out_copy(k, t, False)
    _out_copy(NSEG - 2, NSEG % 2, True)
    _out_copy(NSEG - 1, (NSEG - 1) % 2, True)


_sc_segment_sum = functools.partial(
    pl.kernel,
    out_type=jax.ShapeDtypeStruct((NC, N_PAD, DIM), jnp.float32),
    mesh=plsc.VectorSubcoreMesh(core_axis_name="c", subcore_axis_name="s"),
    compiler_params=pltpu.CompilerParams(use_tc_tiling_on_sc=False),
    scratch_types=[
        pltpu.VMEM((CHUNK, DIM), jnp.float32),
        pltpu.VMEM((CHUNK, DIM), jnp.float32),
        pltpu.VMEM((CHUNK, PDIM), jnp.int32),
        pltpu.VMEM((CHUNK, PDIM), jnp.int32),
        pltpu.VMEM((3, CHUNK), jnp.int32),
        pltpu.VMEM((3, CHUNK), jnp.int32),
        pltpu.VMEM((3, CHUNK), jnp.int32),
        pltpu.VMEM((3, CHUNK), jnp.int32),
        pltpu.SemaphoreType.DMA,
        pltpu.SemaphoreType.DMA,
        pltpu.SemaphoreType.DMA,
        pltpu.SemaphoreType.DMA,
        pltpu.SemaphoreType.DMA,
        pltpu.SemaphoreType.DMA,
        pltpu.SemaphoreType.DMA,
        pltpu.SemaphoreType.DMA,
        pltpu.VMEM_SHARED((N_PAD, DIM), jnp.float32),
    ],
)(_sc_body)


def _tc_body(f_ref, p0_ref, p1_ref, w1_ref, w2_ref, b1_ref, b2_ref, o_ref):
    f = f_ref[...]
    h = p0_ref[...] + p1_ref[...]
    a = lax.dot_general(f + h, w1_ref[...], (((1,), (1,)), ((), ())),
                        preferred_element_type=jnp.float32)
    b = lax.dot_general(f * h, w2_ref[...], (((1,), (1,)), ((), ())),
                        preferred_element_type=jnp.float32)
    x = a + b + b1_ref[...] + b2_ref[...]
    o_ref[...] = jnp.where(x > 0, x, 0.01 * x)


def _tc_mlp(features, p0, p1, W1_w, W2_w, b1, b2):
    block = 2000
    grid = N_NODES // block
    row_spec = pl.BlockSpec((block, DIM), lambda i: (i, 0))
    full_spec = pl.BlockSpec((DIM, DIM), lambda i: (0, 0))
    bias_spec = pl.BlockSpec((1, DIM), lambda i: (0, 0))
    return pl.pallas_call(
        _tc_body,
        grid=(grid,),
        in_specs=[row_spec, row_spec, row_spec, full_spec, full_spec,
                  bias_spec, bias_spec],
        out_specs=row_spec,
        out_shape=jax.ShapeDtypeStruct((N_NODES, DIM), jnp.float32),
    )(features, p0, p1, W1_w, W2_w, b1, b2)


def kernel(features, target, neighbor, values, W1_w, W1_b, W2_w, W2_b):
    pad = E_PAD - N_EDGES
    nbr = jnp.concatenate(
        [neighbor.astype(jnp.int32), jnp.zeros((pad,), jnp.int32)]
    ).reshape(E_PAD // CHUNK, CHUNK)
    tgt = jnp.concatenate(
        [target.astype(jnp.int32), jnp.zeros((pad,), jnp.int32)]
    ).reshape(E_PAD // CHUNK, CHUNK)
    val = lax.bitcast_convert_type(
        jnp.concatenate(
            [values.astype(jnp.float32), jnp.zeros((pad,), jnp.float32)]
        ), jnp.int32
    ).reshape(E_PAD // CHUNK, CHUNK)
    idx = jnp.stack([nbr, tgt, val], axis=1)

    featb = features.astype(jnp.bfloat16).reshape(N_NODES, 4, 2, 16)
    featp = lax.bitcast_convert_type(
        featb.transpose(0, 1, 3, 2), jnp.int32).reshape(N_NODES, PDIM)

    partials = _sc_segment_sum(idx, featp)
    return _tc_mlp(features, partials[0, :N_NODES], partials[1, :N_NODES],
                   W1_w, W2_w, W1_b.reshape(1, DIM), W2_b.reshape(1, DIM))
